# Initial kernel scaffold; baseline (speedup 1.0000x reference)
import jax
import jax.numpy as jnp
from jax.experimental import pallas as pl


def kernel(x, edge_index, batch, w_embed, b_embed, bn0_g, bn0_b, gat_w, gat_att_src, gat_att_dst, gat_b, bn_g, bn_b, cw1, cb1, cw2, cb2, cw3, cb3, cw4, cb4):
    raise NotImplementedError("write your pallas kernel here")


# speedup vs baseline: 1.0645x; 1.0645x over previous
import functools

import jax
import jax.numpy as jnp
import numpy as np
from jax.experimental import pallas as pl

N = 10000
E = 320000
G = 64
DIN = 48
HID = 256
HEADS = 8
C = 32
L = 4
INV_STD = 1.0 / np.sqrt(1.0 + 1e-5)


def _elu(x):
    return jnp.where(x > 0, x, jnp.exp(jnp.minimum(x, 0.0)) - 1.0)


def _bdot(a, b):
    return jnp.dot(a.astype(jnp.bfloat16), b.astype(jnp.bfloat16),
                   preferred_element_type=jnp.float32)


def _embed_body(x_ref, w_ref, b_ref, g_ref, bb_ref, o_ref):
    h = _bdot(x_ref[:], w_ref[:])
    h = (h + b_ref[:]) * INV_STD * g_ref[:] + bb_ref[:]
    o_ref[:] = _elu(h)


def _proj_body(h_ref, w_ref, as_ref, ad_ref, hp_ref, asrc_ref, adst_ref):
    hp = _bdot(h_ref[:], w_ref[:])
    hp_ref[:] = hp
    asrc_ref[:] = jnp.dot(hp, as_ref[:], preferred_element_type=jnp.float32)
    adst_ref[:] = jnp.dot(hp, ad_ref[:], preferred_element_type=jnp.float32)


def _post_body(add_res, agg_ref, b_ref, g_ref, bb_ref, prev_ref, o_ref):
    h = _elu(agg_ref[:] + b_ref[:])
    h = h * INV_STD * g_ref[:] + bb_ref[:]
    if add_res:
        h = h + prev_ref[:]
    o_ref[:] = h


def _pool_mlp_body(h_ref, brow_ref, bcol_ref, w1_ref, b1_ref, w2_ref, b2_ref,
                   w3_ref, b3_ref, w4_ref, b4_ref, out_ref, g_ref):
    h = h_ref[:]
    brow = brow_ref[:]
    bcol = bcol_ref[:]
    gi = jax.lax.broadcasted_iota(jnp.int32, (G, N), 0)
    onehot = (gi == brow).astype(jnp.float32)
    g_add = jnp.dot(onehot, h, preferred_element_type=jnp.float32)
    cnt = jnp.sum(onehot, axis=1, keepdims=True)
    g_mean = g_add / jnp.maximum(cnt, 1.0)

    def body(g, carry):
        mask = bcol == g
        vals = jnp.where(mask, h, -jnp.inf)
        row = jnp.max(vals, axis=0, keepdims=True)
        g_ref[pl.ds(g, 1), HID:2 * HID] = row
        return carry

    jax.lax.fori_loop(0, G, body, 0)
    g_max = g_ref[:, HID:2 * HID]

    g_ref[:, 0:HID] = g_mean
    g_ref[:, 2 * HID:3 * HID] = g_add

    z = (_bdot(g_mean, w1_ref[0:HID, :])
         + _bdot(g_max, w1_ref[HID:2 * HID, :])
         + _bdot(g_add, w1_ref[2 * HID:3 * HID, :])
         + b1_ref[:])
    z = _elu(z)
    z = _elu(_bdot(z, w2_ref[:]) + b2_ref[:])
    z = _elu(_bdot(z, w3_ref[:]) + b3_ref[:])
    out_ref[:] = _bdot(z, w4_ref[:]) + b4_ref[:]


def _f32(shape):
    return jax.ShapeDtypeStruct(shape, jnp.float32)


@jax.jit
def kernel(x, edge_index, batch, w_embed, b_embed, bn0_g, bn0_b, gat_w,
           gat_att_src, gat_att_dst, gat_b, bn_g, bn_b, cw1, cb1, cw2, cb2,
           cw3, cb3, cw4, cb4):
    loop = jnp.arange(N, dtype=edge_index.dtype)
    src = jnp.concatenate([edge_index[0], loop])
    dst = jnp.concatenate([edge_index[1], loop])

    h = pl.pallas_call(_embed_body, out_shape=_f32((N, HID)))(
        x, w_embed, b_embed.reshape(1, HID), bn0_g.reshape(1, HID),
        bn0_b.reshape(1, HID))

    eye = jnp.eye(HEADS, dtype=jnp.float32)
    for i in range(L):
        h_prev = h
        As = (gat_att_src[i][:, :, None] * eye[:, None, :]).reshape(HID, HEADS)
        Ad = (gat_att_dst[i][:, :, None] * eye[:, None, :]).reshape(HID, HEADS)
        hp, asrc, adst = pl.pallas_call(
            _proj_body,
            out_shape=(_f32((N, HID)), _f32((N, HEADS)), _f32((N, HEADS))),
        )(h, gat_w[i], As, Ad)

        alpha = jax.nn.leaky_relu(asrc[src] + adst[dst], 0.2)
        m = jax.ops.segment_max(alpha, dst, num_segments=N)
        m = jnp.where(jnp.isfinite(m), m, 0.0)
        ex = jnp.exp(alpha - m[dst])
        den = jax.ops.segment_sum(ex, dst, num_segments=N)
        coef = ex / (den[dst] + 1e-16)
        msg = coef[:, :, None] * hp.reshape(N, HEADS, C)[src]
        agg = jax.ops.segment_sum(msg, dst, num_segments=N).reshape(N, HID)

        h = pl.pallas_call(
            functools.partial(_post_body, i > 0),
            out_shape=_f32((N, HID)),
        )(agg, gat_b[i].reshape(1, HID), bn_g[i].reshape(1, HID),
          bn_b[i].reshape(1, HID), h_prev)

    out, g = pl.pallas_call(
        _pool_mlp_body,
        out_shape=(_f32((G, 2)), _f32((G, 3 * HID))),
    )(h, batch.reshape(1, N), batch.reshape(N, 1),
      cw1, cb1.reshape(1, 2 * HID), cw2, cb2.reshape(1, HID),
      cw3, cb3.reshape(1, HID // 2), cw4, cb4.reshape(1, 2))
    return (out, g)
